# Initial kernel scaffold; baseline (speedup 1.0000x reference)
import functools

import jax
import jax.numpy as jnp
from jax import lax
from jax.experimental import pallas as pl
from jax.experimental.pallas import tpu as pltpu
from jax.experimental.pallas import tpu_sc as plsc

EMBED_DIM = 128
L_WORD = 50
L_RELA = 8
WPAD = 104
EPS = 1e-8
LANES = 16
NCHUNK = EMBED_DIM // LANES


def _rowsum_pair(wbuf, n):
    zero = jnp.zeros((LANES,), jnp.float32)

    def body(r, carry):
        qa, ra = carry
        qa = tuple(qa[j] + wbuf[r, pl.ds(j * LANES, LANES)] for j in range(NCHUNK))
        ra = tuple(ra[j] + wbuf[r + n, pl.ds(j * LANES, LANES)] for j in range(NCHUNK))
        return (qa, ra)

    init = (tuple(zero for _ in range(NCHUNK)), tuple(zero for _ in range(NCHUNK)))
    return lax.fori_loop(0, n, body, init, unroll=2)


def _sc_body(widx_hbm, ridx_hbm, wtab, rtab, out_hbm,
             widx_v, ridx_v, wrow0, wrow1, rrow0, rrow1, score_v,
             sw0, sw1, sr0, sr1):
    nc = 2
    wid = lax.axis_index("s") * nc + lax.axis_index("c")
    bpw = 128
    base = wid * bpw

    pltpu.sync_copy(widx_hbm.at[pl.ds(base, bpw)], widx_v)
    pltpu.sync_copy(ridx_hbm.at[pl.ds(base, bpw)], ridx_v)

    wrows = (wrow0, wrow1)
    rrows = (rrow0, rrow1)
    sws = (sw0, sw1)
    srs = (sr0, sr1)

    def issue(e, b):
        pltpu.async_copy(wtab.at[widx_v.at[e]], wrows[b], sws[b])
        pltpu.async_copy(rtab.at[ridx_v.at[e]], rrows[b], srs[b])

    def wait(e, b):
        pltpu.make_async_copy(wtab.at[widx_v.at[e]], wrows[b], sws[b]).wait()
        pltpu.make_async_copy(rtab.at[ridx_v.at[e]], rrows[b], srs[b]).wait()

    issue(0, 0)

    lane_iota = lax.iota(jnp.int32, LANES)

    def compute_elem(wbuf, rbuf):
        qsum, rtsum = _rowsum_pair(wbuf, L_WORD)
        dv = jnp.zeros((LANES,), jnp.float32)
        n1 = jnp.zeros((LANES,), jnp.float32)
        n2 = jnp.zeros((LANES,), jnp.float32)
        for j in range(NCHUNK):
            rr = rbuf[0, pl.ds(j * LANES, LANES)]
            for r in range(1, L_RELA):
                rr = rr + rbuf[r, pl.ds(j * LANES, LANES)]
            q = qsum[j] * (1.0 / L_WORD)
            rm = rtsum[j] * (1.0 / L_WORD) + rr * (1.0 / L_RELA)
            dv = dv + q * rm
            n1 = n1 + q * q
            n2 = n2 + rm * rm
        return jnp.sum(dv), jnp.sum(n1), jnp.sum(n2)

    @pl.loop(0, 8)
    def _group(g):
        def pbody(p, carry):
            dacc, n1acc, n2acc = carry
            for b in range(2):
                k = p * 2 + b
                e = g * 16 + k
                nxt = e + 1

                @pl.when(nxt < bpw)
                def _():
                    issue(jnp.minimum(nxt, bpw - 1), 1 - b)

                wait(e, b)
                d_s, n1_s, n2_s = compute_elem(wrows[b], rrows[b])
                sel = lane_iota == k
                dacc = jnp.where(sel, d_s, dacc)
                n1acc = jnp.where(sel, n1_s, n1acc)
                n2acc = jnp.where(sel, n2_s, n2acc)
            return (dacc, n1acc, n2acc)

        zero = jnp.zeros((LANES,), jnp.float32)
        dacc, n1acc, n2acc = lax.fori_loop(0, 8, pbody, (zero, zero, zero))

        prod = n1acc * n2acc
        yi = (plsc.bitcast(prod, jnp.int32) >> 1) + jnp.int32(0x1FBD1DF5)
        y = plsc.bitcast(yi, jnp.float32)
        for _ in range(3):
            y = 0.5 * (y + prod / y)
        score = dacc / jnp.maximum(y, EPS)
        score_v[pl.ds(g * 16, LANES)] = score

    pltpu.sync_copy(score_v, out_hbm.at[pl.ds(base, bpw)])


@functools.cache
def _build(batch):
    mesh = plsc.VectorSubcoreMesh(core_axis_name="c", subcore_axis_name="s")
    return pl.kernel(
        _sc_body,
        out_type=jax.ShapeDtypeStruct((batch,), jnp.float32),
        mesh=mesh,
        scratch_types=[
            pltpu.VMEM((128, WPAD), jnp.int32),
            pltpu.VMEM((128, L_RELA), jnp.int32),
            pltpu.VMEM((WPAD, EMBED_DIM), jnp.float32),
            pltpu.VMEM((WPAD, EMBED_DIM), jnp.float32),
            pltpu.VMEM((L_RELA, EMBED_DIM), jnp.float32),
            pltpu.VMEM((L_RELA, EMBED_DIM), jnp.float32),
            pltpu.VMEM((128,), jnp.float32),
            pltpu.SemaphoreType.DMA,
            pltpu.SemaphoreType.DMA,
            pltpu.SemaphoreType.DMA,
            pltpu.SemaphoreType.DMA,
        ],
    )


def kernel(ques_x, rela_text_x, rela_x, word_emb, rela_emb):
    batch = ques_x.shape[0]
    pad = jnp.zeros((batch, WPAD - 2 * L_WORD), jnp.int32)
    widx = jnp.concatenate(
        [ques_x.astype(jnp.int32), rela_text_x.astype(jnp.int32), pad], axis=1)
    ridx = rela_x.astype(jnp.int32)
    return _build(batch)(widx, ridx, word_emb, rela_emb)


# speedup vs baseline: 2.5136x; 2.5136x over previous
import functools

import jax
import jax.numpy as jnp
from jax import lax
from jax.experimental import pallas as pl
from jax.experimental.pallas import tpu as pltpu
from jax.experimental.pallas import tpu_sc as plsc

EMBED_DIM = 128
L_WORD = 50
L_RELA = 8
WPAD = 104
EPS = 1e-8
LANES = 16
NCHUNK = EMBED_DIM // LANES


def _rowsum_pair(wbuf, n):
    zero = jnp.zeros((LANES,), jnp.float32)

    def body(r, carry):
        qa, ra = carry
        qa = tuple(qa[j] + wbuf[r, pl.ds(j * LANES, LANES)] for j in range(NCHUNK))
        ra = tuple(ra[j] + wbuf[r + n, pl.ds(j * LANES, LANES)] for j in range(NCHUNK))
        return (qa, ra)

    init = (tuple(zero for _ in range(NCHUNK)), tuple(zero for _ in range(NCHUNK)))
    return lax.fori_loop(0, n, body, init, unroll=2)


def _sc_body(widx_hbm, ridx_hbm, wtab, rtab, out_hbm,
             widx_v, ridx_v, wrow0, wrow1, rrow0, rrow1, score_v,
             sw0, sw1, sr0, sr1):
    nc = 2
    wid = lax.axis_index("s") * nc + lax.axis_index("c")
    bpw = 128
    base = wid * bpw

    pltpu.sync_copy(widx_hbm.at[pl.ds(base, bpw)], widx_v)
    pltpu.sync_copy(ridx_hbm.at[pl.ds(base, bpw)], ridx_v)

    wrows = (wrow0, wrow1)
    rrows = (rrow0, rrow1)
    sws = (sw0, sw1)
    srs = (sr0, sr1)

    def issue(e, b):
        pltpu.async_copy(wtab.at[widx_v.at[e]], wrows[b], sws[b])
        pltpu.async_copy(rtab.at[ridx_v.at[e]], rrows[b], srs[b])

    def wait(e, b):
        pltpu.make_async_copy(wtab.at[widx_v.at[e]], wrows[b], sws[b]).wait()
        pltpu.make_async_copy(rtab.at[ridx_v.at[e]], rrows[b], srs[b]).wait()

    issue(0, 0)

    lane_iota = lax.iota(jnp.int32, LANES)

    def compute_elem(wbuf, rbuf):
        qsum, rtsum = _rowsum_pair(wbuf, L_WORD)
        dv = jnp.zeros((LANES,), jnp.float32)
        n1 = jnp.zeros((LANES,), jnp.float32)
        n2 = jnp.zeros((LANES,), jnp.float32)
        for j in range(NCHUNK):
            rr = rbuf[0, pl.ds(j * LANES, LANES)]
            for r in range(1, L_RELA):
                rr = rr + rbuf[r, pl.ds(j * LANES, LANES)]
            q = qsum[j] * (1.0 / L_WORD)
            rm = rtsum[j] * (1.0 / L_WORD) + rr * (1.0 / L_RELA)
            dv = dv + q * rm
            n1 = n1 + q * q
            n2 = n2 + rm * rm
        for s in (8, 4, 2, 1):
            idx = lane_iota ^ s
            dv = dv + dv.at[idx].get(mode="promise_in_bounds")
            n1 = n1 + n1.at[idx].get(mode="promise_in_bounds")
            n2 = n2 + n2.at[idx].get(mode="promise_in_bounds")
        return dv, n1, n2

    @pl.loop(0, 8)
    def _group(g):
        def pbody(p, carry):
            dacc, n1acc, n2acc = carry
            for b in range(2):
                k = p * 2 + b
                e = g * 16 + k
                nxt = e + 1

                @pl.when(nxt < bpw)
                def _():
                    issue(jnp.minimum(nxt, bpw - 1), 1 - b)

                wait(e, b)
                d_v, n1_v, n2_v = compute_elem(wrows[b], rrows[b])
                sel = lane_iota == k
                dacc = jnp.where(sel, d_v, dacc)
                n1acc = jnp.where(sel, n1_v, n1acc)
                n2acc = jnp.where(sel, n2_v, n2acc)
            return (dacc, n1acc, n2acc)

        zero = jnp.zeros((LANES,), jnp.float32)
        dacc, n1acc, n2acc = lax.fori_loop(0, 8, pbody, (zero, zero, zero))

        prod = n1acc * n2acc
        yi = (lax.bitcast_convert_type(prod, jnp.int32) >> 1) + jnp.int32(0x1FBD1DF5)
        y = lax.bitcast_convert_type(yi, jnp.float32)
        for _ in range(3):
            y = 0.5 * (y + prod / y)
        score = dacc / jnp.maximum(y, EPS)
        score_v[pl.ds(g * 16, LANES)] = score

    pltpu.sync_copy(score_v, out_hbm.at[pl.ds(base, bpw)])


@functools.cache
def _build(batch):
    mesh = plsc.VectorSubcoreMesh(core_axis_name="c", subcore_axis_name="s")
    return pl.kernel(
        _sc_body,
        out_type=jax.ShapeDtypeStruct((batch,), jnp.float32),
        mesh=mesh,
        scratch_types=[
            pltpu.VMEM((128, WPAD), jnp.int32),
            pltpu.VMEM((128, L_RELA), jnp.int32),
            pltpu.VMEM((WPAD, EMBED_DIM), jnp.float32),
            pltpu.VMEM((WPAD, EMBED_DIM), jnp.float32),
            pltpu.VMEM((L_RELA, EMBED_DIM), jnp.float32),
            pltpu.VMEM((L_RELA, EMBED_DIM), jnp.float32),
            pltpu.VMEM((128,), jnp.float32),
            pltpu.SemaphoreType.DMA,
            pltpu.SemaphoreType.DMA,
            pltpu.SemaphoreType.DMA,
            pltpu.SemaphoreType.DMA,
        ],
    )


def kernel(ques_x, rela_text_x, rela_x, word_emb, rela_emb):
    batch = ques_x.shape[0]
    pad = jnp.zeros((batch, WPAD - 2 * L_WORD), jnp.int32)
    widx = jnp.concatenate(
        [ques_x.astype(jnp.int32), rela_text_x.astype(jnp.int32), pad], axis=1)
    ridx = rela_x.astype(jnp.int32)
    return _build(batch)(widx, ridx, word_emb, rela_emb)
